# Initial kernel scaffold; baseline (speedup 1.0000x reference)
import jax
import jax.numpy as jnp
from jax.experimental import pallas as pl


def kernel(input_ids, embed_tokens):
    raise NotImplementedError("write your pallas kernel here")


# speedup vs baseline: 1.4800x; 1.4800x over previous
import functools

import jax
import jax.numpy as jnp
from jax import lax
from jax.experimental import pallas as pl
from jax.experimental.pallas import tpu as pltpu
from jax.experimental.pallas import tpu_sc as plsc

HIDDEN = 4096
NC, NS = 2, 16
NW = NC * NS
K = 8


@functools.partial(jax.jit, static_argnames=("batch",))
def _embedding_lookup(ids, table, *, batch):
    b_per_w = batch // NW
    nsteps = b_per_w // K
    mesh = plsc.VectorSubcoreMesh(
        core_axis_name="c", subcore_axis_name="s", num_cores=NC, num_subcores=NS
    )

    @functools.partial(
        pl.kernel,
        out_type=jax.ShapeDtypeStruct((batch, HIDDEN), jnp.float32),
        mesh=mesh,
        scratch_types=[
            pltpu.VMEM((nsteps, K), jnp.int32),
            pltpu.VMEM((K, HIDDEN), jnp.float32),
            pltpu.SemaphoreType.DMA,
        ],
    )
    def body(table_hbm, ids_hbm, out_hbm, idx_v, rows_v, sem):
        wid = lax.axis_index("s") * NC + lax.axis_index("c")
        base = wid * b_per_w
        pltpu.sync_copy(ids_hbm.at[wid], idx_v)

        @pl.loop(0, nsteps)
        def _(i):
            pltpu.async_copy(table_hbm.at[idx_v.at[i]], rows_v, sem).wait()
            pltpu.sync_copy(rows_v, out_hbm.at[pl.ds(base + i * K, K)])

    return body(table, ids)


def kernel(input_ids, embed_tokens):
    batch = input_ids.size
    ids = input_ids.reshape(NW, batch // (NW * K), K).astype(jnp.int32)
    out = _embedding_lookup(ids, embed_tokens, batch=batch)
    return out.reshape(*input_ids.shape, HIDDEN)


# speedup vs baseline: 1.7625x; 1.1909x over previous
import functools

import jax
import jax.numpy as jnp
from jax import lax
from jax.experimental import pallas as pl
from jax.experimental.pallas import tpu as pltpu
from jax.experimental.pallas import tpu_sc as plsc

HIDDEN = 4096
NC, NS = 2, 16
NW = NC * NS
K = 8
NBUF = 2


@functools.partial(jax.jit, static_argnames=("batch",))
def _embedding_lookup(ids, table, *, batch):
    b_per_w = batch // NW
    nsteps = b_per_w // K
    mesh = plsc.VectorSubcoreMesh(
        core_axis_name="c", subcore_axis_name="s", num_cores=NC, num_subcores=NS
    )

    @functools.partial(
        pl.kernel,
        out_type=jax.ShapeDtypeStruct((batch, HIDDEN), jnp.float32),
        mesh=mesh,
        scratch_types=[
            pltpu.VMEM((nsteps, K), jnp.int32),
            [pltpu.VMEM((K, HIDDEN), jnp.float32) for _ in range(NBUF)],
            [pltpu.SemaphoreType.DMA for _ in range(NBUF)],
            [pltpu.SemaphoreType.DMA for _ in range(NBUF)],
        ],
    )
    def body(table_hbm, ids_hbm, out_hbm, idx_v, rows, gsem, ssem):
        wid = lax.axis_index("s") * NC + lax.axis_index("c")
        base = wid * b_per_w
        pltpu.sync_copy(ids_hbm.at[wid], idx_v)

        def gather(s, b):
            pltpu.make_async_copy(table_hbm.at[idx_v.at[s]], rows[b], gsem[b]).start()

        def store(s, b):
            dst = out_hbm.at[pl.ds(base + s * K, K)]
            pltpu.make_async_copy(rows[b], dst, ssem[b]).start()

        for b in range(NBUF):
            gather(b, b)

        @pl.loop(0, nsteps - NBUF, step=NBUF)
        def _(i):
            for b in range(NBUF):
                s = i + b
                pltpu.make_async_copy(table_hbm.at[idx_v.at[s]], rows[b], gsem[b]).wait()
                store(s, b)
                pltpu.make_async_copy(rows[b], out_hbm.at[pl.ds(base + s * K, K)], ssem[b]).wait()
                gather(s + NBUF, b)

        for b in range(NBUF):
            s = nsteps - NBUF + b
            pltpu.make_async_copy(table_hbm.at[idx_v.at[s]], rows[b], gsem[b]).wait()
            store(s, b)
        for b in range(NBUF):
            s = nsteps - NBUF + b
            pltpu.make_async_copy(rows[b], out_hbm.at[pl.ds(base + s * K, K)], ssem[b]).wait()

    return body(table, ids)


def kernel(input_ids, embed_tokens):
    batch = input_ids.size
    ids = input_ids.reshape(NW, batch // (NW * K), K).astype(jnp.int32)
    out = _embedding_lookup(ids, embed_tokens, batch=batch)
    return out.reshape(*input_ids.shape, HIDDEN)
